# baseline (speedup 1.0000x reference)
import jax
import jax.numpy as jnp
import numpy as np
from jax.experimental import pallas as pl
from jax.experimental.pallas import tpu as pltpu


def _decoder_body(T, V, Vp, DIN, H, ET,
                  w_ref, ins_ref, sq_ref, sqb_ref, inw_ref, o3w_ref,
                  bias_ref, b3_ref, out_ref):
    P = V * Vp
    H2 = 2 * H
    f32 = jnp.float32
    bf16 = jnp.bfloat16

    inv_norm = 1.0 / ((ET - 1.0) * (V - 1.0))
    dot = lambda a, b: jnp.dot(a, b, preferred_element_type=f32)

    i_hr = 3 * ET

    wp = [jnp.concatenate(
              [jnp.broadcast_to(w_ref[bb, :, et:et + 1], (P, H))
               for bb in range(2)], axis=1)
          for et in range(ET)]

    ins = ins_ref[0]
    hidden = jnp.zeros((Vp, H2), dtype=f32)

    for t in range(T):
        m2w = jnp.zeros((P, H2), dtype=f32)
        hid_bf = hidden.astype(bf16)
        for et in range(1, ET):
            s_part = dot(hid_bf, sqb_ref[et])
            a_part = dot(hid_bf, sqb_ref[ET + et]) + bias_ref[et]
            pre = s_part[:V][:, None, :] + a_part[None, :, :]
            m = jnp.tanh(pre).reshape(P, H2)
            m2 = jnp.tanh(dot(m.astype(bf16), sqb_ref[2 * ET + et])
                          + bias_ref[ET + et])
            m2w = m2w + m2 * wp[et]
        agg = jnp.sum(m2w.reshape(V, Vp, H2), axis=0) * inv_norm

        inp_r = dot(ins, inw_ref[0]) + bias_ref[2 * ET]
        inp_i = dot(ins, inw_ref[1]) + bias_ref[2 * ET + 1]
        inp_n = dot(ins, inw_ref[2]) + bias_ref[2 * ET + 2]
        r = jax.nn.sigmoid(inp_r + dot(agg, sq_ref[i_hr]))
        ig = jax.nn.sigmoid(inp_i + dot(agg, sq_ref[i_hr + 1]))
        n = jnp.tanh(inp_n + r * dot(agg, sq_ref[i_hr + 2]))
        hidden = (1.0 - ig) * n + ig * hidden

        p = jax.nn.relu(dot(hidden, sq_ref[i_hr + 3]) + bias_ref[2 * ET + 3])
        p = jax.nn.relu(dot(p, sq_ref[i_hr + 4]) + bias_ref[2 * ET + 4])
        p = dot(p, o3w_ref[...]) + b3_ref[0]
        pred = ins + p
        out_ref[0, t] = pred[:V, :DIN]
        out_ref[1, t] = pred[:V, DIN:]
        ins = pred


def kernel(inputs, sampled_edges, msg_fc1_w, msg_fc1_b, msg_fc2_w,
           msg_fc2_b, hidden_r_w, hidden_i_w, hidden_h_w, input_r_w,
           input_r_b, input_i_w, input_i_b, input_n_w, input_n_b,
           out_fc1_w, out_fc1_b, out_fc2_w, out_fc2_b, out_fc3_w,
           out_fc3_b):
    B, T, V, DIN = inputs.shape
    H = hidden_r_w.shape[0]
    ET = msg_fc1_w.shape[0]
    Vp = (V + 7) // 8 * 8
    P = V * Vp
    NB = 2

    se4 = sampled_edges.reshape(B, V, V - 1, ET)
    se_pad = jnp.pad(se4, ((0, 0), (0, 0), (0, Vp - (V - 1)), (0, 0)))
    se_shift = jnp.roll(se_pad, 1, axis=2)
    jj = np.arange(Vp)[None, :]
    ii = np.arange(V)[:, None]
    mask_lt = jnp.asarray((jj < ii)[None, :, :, None])
    mask_gt = jnp.asarray(((jj > ii) & (jj < V))[None, :, :, None])
    w4 = (jnp.where(mask_lt, se_pad, 0.0) +
          jnp.where(mask_gt, se_shift, 0.0))
    w_dense = w4.reshape(B, P, ET)

    ins0 = jnp.pad(inputs[:, 0], ((0, 0), (0, Vp - V), (0, 0)))
    ins0 = ins0.reshape(B // NB, NB, Vp, DIN).transpose(0, 2, 1, 3)
    ins0 = ins0.reshape(B // NB, Vp, NB * DIN)

    def bd(ms):
        z = jnp.zeros_like(ms)
        top = jnp.concatenate([ms, z], axis=2)
        bot = jnp.concatenate([z, ms], axis=2)
        return jnp.concatenate([top, bot], axis=1)

    sq = jnp.concatenate(
        [msg_fc1_w[:, H:, :], msg_fc1_w[:, :H, :], msg_fc2_w,
         hidden_r_w[None], hidden_i_w[None], hidden_h_w[None],
         out_fc1_w[None], out_fc2_w[None]], axis=0)
    sq_bd = bd(sq)
    inw_bd = bd(jnp.stack([input_r_w, input_i_w, input_n_w]))
    o3w_bd = bd(out_fc3_w[None])[0]
    bias = jnp.concatenate(
        [msg_fc1_b, msg_fc2_b,
         jnp.stack([input_r_b, input_i_b, input_n_b,
                    out_fc1_b, out_fc2_b])], axis=0)
    bias2 = jnp.tile(bias, (1, NB))
    b3 = jnp.tile(out_fc3_b.reshape(1, -1), (1, NB))

    def body(*refs):
        _decoder_body(T, V, Vp, DIN, H, ET, *refs)

    NSQ = 3 * ET + 5
    H2 = 2 * H
    D2 = 2 * DIN

    out = pl.pallas_call(
        body,
        grid=(B // NB,),
        in_specs=[
            pl.BlockSpec((NB, P, ET), lambda b: (b, 0, 0)),
            pl.BlockSpec((1, Vp, D2), lambda b: (b, 0, 0)),
            pl.BlockSpec((NSQ, H2, H2), lambda b: (0, 0, 0)),
            pl.BlockSpec((NSQ, H2, H2), lambda b: (0, 0, 0)),
            pl.BlockSpec((3, D2, H2), lambda b: (0, 0, 0)),
            pl.BlockSpec((H2, D2), lambda b: (0, 0)),
            pl.BlockSpec((2 * ET + 5, H2), lambda b: (0, 0)),
            pl.BlockSpec((1, D2), lambda b: (0, 0)),
        ],
        out_specs=pl.BlockSpec((NB, T, V, DIN), lambda b: (b, 0, 0, 0)),
        out_shape=jax.ShapeDtypeStruct((B, T, V, DIN), jnp.float32),
        compiler_params=pltpu.CompilerParams(
            dimension_semantics=("parallel",)),
    )(w_dense, ins0, sq_bd, sq_bd.astype(jnp.bfloat16), inw_bd, o3w_bd,
      bias2, b3)
    return out


# speedup vs baseline: 1.2229x; 1.2229x over previous
import jax
import jax.numpy as jnp
import numpy as np
from jax.experimental import pallas as pl
from jax.experimental.pallas import tpu as pltpu


def _decoder_body(T, V, Vp, DIN, H, ET,
                  w_ref, ins_ref, fc1_ref, w2_ref, gh_ref, gi_ref,
                  o12_ref, o3w_ref, bias_ref, b3_ref, out_ref):
    P = V * Vp
    H2 = 2 * H
    NT = ET - 1
    f32 = jnp.float32

    dot = lambda a, b: jnp.dot(a, b, preferred_element_type=f32)
    sl = lambda x, k: x[:, k * H2:(k + 1) * H2]


    wp = [jnp.concatenate(
              [jnp.broadcast_to(w_ref[bb, :, et:et + 1], (P, H))
               for bb in range(2)], axis=1)
          for et in range(1, ET)]

    ins = ins_ref[0]
    hidden = jnp.zeros((Vp, H2), dtype=f32)

    for t in range(T):
        sa_all = dot(hidden, fc1_ref[...])
        m2w = jnp.zeros((P, H2), dtype=f32)
        for k in range(NT):
            s_part = sl(sa_all, k)
            a_part = sl(sa_all, NT + k) + bias_ref[k + 1]
            pre = s_part[:V][:, None, :] + a_part[None, :, :]
            m = jnp.tanh(pre).reshape(P, H2)
            m2 = jnp.tanh(dot(m, w2_ref[k]) + bias_ref[ET + k + 1])
            m2w = m2w + m2 * wp[k]
        agg = jnp.sum(m2w.reshape(V, Vp, H2), axis=0)

        g_in = dot(ins, gi_ref[...])
        g_h = dot(agg, gh_ref[...])
        r = jax.nn.sigmoid(sl(g_in, 0) + bias_ref[2 * ET] + sl(g_h, 0))
        ig = jax.nn.sigmoid(sl(g_in, 1) + bias_ref[2 * ET + 1] + sl(g_h, 1))
        n = jnp.tanh(sl(g_in, 2) + bias_ref[2 * ET + 2] + r * sl(g_h, 2))
        hidden = (1.0 - ig) * n + ig * hidden

        p = jax.nn.relu(dot(hidden, o12_ref[0]) + bias_ref[2 * ET + 3])
        p = jax.nn.relu(dot(p, o12_ref[1]) + bias_ref[2 * ET + 4])
        p = dot(p, o3w_ref[...]) + b3_ref[0]
        pred = ins + p
        out_ref[0, t] = pred[:V, :DIN]
        out_ref[1, t] = pred[:V, DIN:]
        ins = pred


def kernel(inputs, sampled_edges, msg_fc1_w, msg_fc1_b, msg_fc2_w,
           msg_fc2_b, hidden_r_w, hidden_i_w, hidden_h_w, input_r_w,
           input_r_b, input_i_w, input_i_b, input_n_w, input_n_b,
           out_fc1_w, out_fc1_b, out_fc2_w, out_fc2_b, out_fc3_w,
           out_fc3_b):
    B, T, V, DIN = inputs.shape
    H = hidden_r_w.shape[0]
    ET = msg_fc1_w.shape[0]
    Vp = (V + 7) // 8 * 8
    P = V * Vp
    NB = 2

    se4 = sampled_edges.reshape(B, V, V - 1, ET)
    se_pad = jnp.pad(se4, ((0, 0), (0, 0), (0, Vp - (V - 1)), (0, 0)))
    se_shift = jnp.roll(se_pad, 1, axis=2)
    jj = np.arange(Vp)[None, :]
    ii = np.arange(V)[:, None]
    mask_lt = jnp.asarray((jj < ii)[None, :, :, None])
    mask_gt = jnp.asarray(((jj > ii) & (jj < V))[None, :, :, None])
    w4 = (jnp.where(mask_lt, se_pad, 0.0) +
          jnp.where(mask_gt, se_shift, 0.0))
    inv_norm = 1.0 / ((ET - 1.0) * (V - 1.0))
    w_dense = w4.reshape(B, P, ET) * inv_norm

    ins0 = jnp.pad(inputs[:, 0], ((0, 0), (0, Vp - V), (0, 0)))
    ins0 = ins0.reshape(B // NB, NB, Vp, DIN).transpose(0, 2, 1, 3)
    ins0 = ins0.reshape(B // NB, Vp, NB * DIN)

    def bd(ms):
        z = jnp.zeros_like(ms)
        top = jnp.concatenate([ms, z], axis=2)
        bot = jnp.concatenate([z, ms], axis=2)
        return jnp.concatenate([top, bot], axis=1)

    NT = ET - 1
    sq = jnp.concatenate(
        [msg_fc1_w[1:, H:, :], msg_fc1_w[1:, :H, :], msg_fc2_w[1:],
         hidden_r_w[None], hidden_i_w[None], hidden_h_w[None],
         out_fc1_w[None], out_fc2_w[None]], axis=0)
    sq_bd = bd(sq)
    fc1_cat = jnp.concatenate(
        [sq_bd[k] for k in range(2 * NT)], axis=1)
    w2_bd = sq_bd[2 * NT:3 * NT]
    gh_cat = jnp.concatenate(
        [sq_bd[3 * NT], sq_bd[3 * NT + 1], sq_bd[3 * NT + 2]], axis=1)
    o12_bd = sq_bd[3 * NT + 3:3 * NT + 5]
    inw_bd = bd(jnp.stack([input_r_w, input_i_w, input_n_w]))
    gi_cat = jnp.concatenate([inw_bd[0], inw_bd[1], inw_bd[2]], axis=1)
    o3w_bd = bd(out_fc3_w[None])[0]
    bias = jnp.concatenate(
        [msg_fc1_b, msg_fc2_b,
         jnp.stack([input_r_b, input_i_b, input_n_b,
                    out_fc1_b, out_fc2_b])], axis=0)
    bias2 = jnp.tile(bias, (1, NB))
    b3 = jnp.tile(out_fc3_b.reshape(1, -1), (1, NB))

    def body(*refs):
        _decoder_body(T, V, Vp, DIN, H, ET, *refs)

    H2 = 2 * H
    D2 = 2 * DIN

    out = pl.pallas_call(
        body,
        grid=(B // NB,),
        in_specs=[
            pl.BlockSpec((NB, P, ET), lambda b: (b, 0, 0)),
            pl.BlockSpec((1, Vp, D2), lambda b: (b, 0, 0)),
            pl.BlockSpec((H2, 2 * NT * H2), lambda b: (0, 0)),
            pl.BlockSpec((NT, H2, H2), lambda b: (0, 0, 0)),
            pl.BlockSpec((H2, 3 * H2), lambda b: (0, 0)),
            pl.BlockSpec((D2, 3 * H2), lambda b: (0, 0)),
            pl.BlockSpec((2, H2, H2), lambda b: (0, 0, 0)),
            pl.BlockSpec((H2, D2), lambda b: (0, 0)),
            pl.BlockSpec((2 * ET + 5, H2), lambda b: (0, 0)),
            pl.BlockSpec((1, D2), lambda b: (0, 0)),
        ],
        out_specs=pl.BlockSpec((NB, T, V, DIN), lambda b: (b, 0, 0, 0)),
        out_shape=jax.ShapeDtypeStruct((B, T, V, DIN), jnp.float32),
        compiler_params=pltpu.CompilerParams(
            dimension_semantics=("parallel",)),
    )(w_dense, ins0, fc1_cat, w2_bd, gh_cat, gi_cat, o12_bd, o3w_bd,
      bias2, b3)
    return out


# speedup vs baseline: 1.2387x; 1.0129x over previous
import jax
import jax.numpy as jnp
import numpy as np
from jax.experimental import pallas as pl
from jax.experimental.pallas import tpu as pltpu


def _decoder_body(T, V, Vp, DIN, H, ET,
                  w_ref, ins_ref, sq_ref, inw_ref, o3w_ref,
                  bias_ref, b3_ref, out_ref):
    P = V * Vp
    H2 = 2 * H
    f32 = jnp.float32

    inv_norm = 1.0 / ((ET - 1.0) * (V - 1.0))
    dot = lambda a, b: jnp.dot(a, b, preferred_element_type=f32)

    i_hr = 3 * ET

    wp = [jnp.concatenate(
              [jnp.broadcast_to(w_ref[bb, :, et:et + 1], (P, H))
               for bb in range(2)], axis=1)
          for et in range(ET)]

    ins = ins_ref[0]
    hidden = jnp.zeros((Vp, H2), dtype=f32)

    for t in range(T):
        m2w = jnp.zeros((P, H2), dtype=f32)
        for et in range(1, ET):
            s_part = dot(hidden, sq_ref[et])
            a_part = dot(hidden, sq_ref[ET + et]) + bias_ref[et]
            pre = s_part[:V][:, None, :] + a_part[None, :, :]
            m = jnp.tanh(pre).reshape(P, H2)
            m2 = jnp.tanh(dot(m, sq_ref[2 * ET + et]) + bias_ref[ET + et])
            m2w = m2w + m2 * wp[et]
        agg = jnp.sum(m2w.reshape(V, Vp, H2), axis=0) * inv_norm

        inp_r = dot(ins, inw_ref[0]) + bias_ref[2 * ET]
        inp_i = dot(ins, inw_ref[1]) + bias_ref[2 * ET + 1]
        inp_n = dot(ins, inw_ref[2]) + bias_ref[2 * ET + 2]
        r = jax.nn.sigmoid(inp_r + dot(agg, sq_ref[i_hr]))
        ig = jax.nn.sigmoid(inp_i + dot(agg, sq_ref[i_hr + 1]))
        n = jnp.tanh(inp_n + r * dot(agg, sq_ref[i_hr + 2]))
        hidden = (1.0 - ig) * n + ig * hidden

        p = jax.nn.relu(dot(hidden, sq_ref[i_hr + 3]) + bias_ref[2 * ET + 3])
        p = jax.nn.relu(dot(p, sq_ref[i_hr + 4]) + bias_ref[2 * ET + 4])
        p = dot(p, o3w_ref[...]) + b3_ref[0]
        pred = ins + p
        out_ref[0, t] = pred[:V, :DIN]
        out_ref[1, t] = pred[:V, DIN:]
        ins = pred


def kernel(inputs, sampled_edges, msg_fc1_w, msg_fc1_b, msg_fc2_w,
           msg_fc2_b, hidden_r_w, hidden_i_w, hidden_h_w, input_r_w,
           input_r_b, input_i_w, input_i_b, input_n_w, input_n_b,
           out_fc1_w, out_fc1_b, out_fc2_w, out_fc2_b, out_fc3_w,
           out_fc3_b):
    B, T, V, DIN = inputs.shape
    H = hidden_r_w.shape[0]
    ET = msg_fc1_w.shape[0]
    Vp = (V + 7) // 8 * 8
    P = V * Vp
    NB = 2

    se4 = sampled_edges.reshape(B, V, V - 1, ET)
    se_pad = jnp.pad(se4, ((0, 0), (0, 0), (0, Vp - (V - 1)), (0, 0)))
    se_shift = jnp.roll(se_pad, 1, axis=2)
    jj = np.arange(Vp)[None, :]
    ii = np.arange(V)[:, None]
    mask_lt = jnp.asarray((jj < ii)[None, :, :, None])
    mask_gt = jnp.asarray(((jj > ii) & (jj < V))[None, :, :, None])
    w4 = (jnp.where(mask_lt, se_pad, 0.0) +
          jnp.where(mask_gt, se_shift, 0.0))
    w_dense = w4.reshape(B, P, ET)

    ins0 = jnp.pad(inputs[:, 0], ((0, 0), (0, Vp - V), (0, 0)))
    ins0 = ins0.reshape(B // NB, NB, Vp, DIN).transpose(0, 2, 1, 3)
    ins0 = ins0.reshape(B // NB, Vp, NB * DIN)

    def bd(ms):
        z = jnp.zeros_like(ms)
        top = jnp.concatenate([ms, z], axis=2)
        bot = jnp.concatenate([z, ms], axis=2)
        return jnp.concatenate([top, bot], axis=1)

    sq = jnp.concatenate(
        [msg_fc1_w[:, H:, :], msg_fc1_w[:, :H, :], msg_fc2_w,
         hidden_r_w[None], hidden_i_w[None], hidden_h_w[None],
         out_fc1_w[None], out_fc2_w[None]], axis=0)
    sq_bd = bd(sq)
    inw_bd = bd(jnp.stack([input_r_w, input_i_w, input_n_w]))
    o3w_bd = bd(out_fc3_w[None])[0]
    bias = jnp.concatenate(
        [msg_fc1_b, msg_fc2_b,
         jnp.stack([input_r_b, input_i_b, input_n_b,
                    out_fc1_b, out_fc2_b])], axis=0)
    bias2 = jnp.tile(bias, (1, NB))
    b3 = jnp.tile(out_fc3_b.reshape(1, -1), (1, NB))

    def body(*refs):
        _decoder_body(T, V, Vp, DIN, H, ET, *refs)

    NSQ = 3 * ET + 5
    H2 = 2 * H
    D2 = 2 * DIN

    out = pl.pallas_call(
        body,
        grid=(B // NB,),
        in_specs=[
            pl.BlockSpec((NB, P, ET), lambda b: (b, 0, 0)),
            pl.BlockSpec((1, Vp, D2), lambda b: (b, 0, 0)),
            pl.BlockSpec((NSQ, H2, H2), lambda b: (0, 0, 0)),
            pl.BlockSpec((3, D2, H2), lambda b: (0, 0, 0)),
            pl.BlockSpec((H2, D2), lambda b: (0, 0)),
            pl.BlockSpec((2 * ET + 5, H2), lambda b: (0, 0)),
            pl.BlockSpec((1, D2), lambda b: (0, 0)),
        ],
        out_specs=pl.BlockSpec((NB, T, V, DIN), lambda b: (b, 0, 0, 0)),
        out_shape=jax.ShapeDtypeStruct((B, T, V, DIN), jnp.float32),
        compiler_params=pltpu.CompilerParams(
            dimension_semantics=("parallel",)),
    )(w_dense, ins0, sq_bd, inw_bd, o3w_bd, bias2, b3)
    return out
